# baseline (speedup 1.0000x reference)
import functools

import jax
import jax.numpy as jnp
from jax import lax
from jax.experimental import pallas as pl
from jax.experimental.pallas import tpu as pltpu
from jax.experimental.pallas import tpu_sc as plsc

BATCH = 16384
EMBED_DIM = 64
SUP_DIM = 2 * EMBED_DIM
LANES = 16
NUM_CORES = 2
NUM_SUBCORES = 16
NUM_WORKERS = NUM_CORES * NUM_SUBCORES
BPW = BATCH // NUM_WORKERS
CHUNK = 128
NCHUNKS = BPW // CHUNK
GROUPS = CHUNK // LANES


def _dot_body(i_hbm, j_hbm, emb_hbm, out_hbm,
              idx_i, idx_j, sup_i, sup_j,
              buf_i0, buf_i1, buf_j0, buf_j1, out_v,
              si0, si1, sj0, sj1):
    c = lax.axis_index("c")
    s = lax.axis_index("s")
    wid = s * NUM_CORES + c

    pltpu.sync_copy(i_hbm.at[wid], idx_i)
    pltpu.sync_copy(j_hbm.at[wid], idx_j)

    for k in range(NCHUNKS):
        for t in range(CHUNK // LANES):
            sl = pl.ds(t * LANES, LANES)
            sup_i[k, sl] = lax.shift_right_logical(idx_i[k, sl], 1)
            sup_j[k, sl] = lax.shift_right_logical(idx_j[k, sl], 1)

    bufs_i = (buf_i0, buf_i1)
    bufs_j = (buf_j0, buf_j1)
    sems_i = (si0, si1)
    sems_j = (sj0, sj1)

    def fire(k):
        b = k % 2
        return (pltpu.async_copy(emb_hbm.at[sup_i.at[k]], bufs_i[b], sems_i[b]),
                pltpu.async_copy(emb_hbm.at[sup_j.at[k]], bufs_j[b], sems_j[b]))

    lane = lax.broadcasted_iota(jnp.int32, (LANES,), 0)
    inflight = fire(0)

    for k in range(NCHUNKS):
        b = k % 2
        for cp in inflight:
            cp.wait()
        if k + 1 < NCHUNKS:
            inflight = fire(k + 1)
        bi, bj = bufs_i[b], bufs_j[b]
        for g in range(GROUPS):
            sl = pl.ds(g * LANES, LANES)
            rows = lane + g * LANES
            hi = (idx_i[k, sl] & 1) * EMBED_DIM
            hj = (idx_j[k, sl] & 1) * EMBED_DIM
            acc = jnp.zeros((LANES,), jnp.float32)
            for d in range(EMBED_DIM):
                vi = plsc.load_gather(bi, [rows, hi + d])
                vj = plsc.load_gather(bj, [rows, hj + d])
                acc = acc + vi * vj
            out_v[pl.ds(k * CHUNK + g * LANES, LANES)] = acc

    pltpu.sync_copy(out_v, out_hbm.at[pl.ds(wid * BPW, BPW)])


@jax.jit
def _sc_dot(i, j, node_emb):
    mesh = plsc.VectorSubcoreMesh(core_axis_name="c", subcore_axis_name="s")
    kfn = pl.kernel(
        _dot_body,
        mesh=mesh,
        compiler_params=pltpu.CompilerParams(
            needs_layout_passes=False, use_tc_tiling_on_sc=True),
        out_type=jax.ShapeDtypeStruct((BATCH,), jnp.float32),
        scratch_types=[
            pltpu.VMEM((NCHUNKS, CHUNK), jnp.int32),
            pltpu.VMEM((NCHUNKS, CHUNK), jnp.int32),
            pltpu.VMEM((NCHUNKS, CHUNK), jnp.int32),
            pltpu.VMEM((NCHUNKS, CHUNK), jnp.int32),
            pltpu.VMEM((CHUNK, SUP_DIM), jnp.float32),
            pltpu.VMEM((CHUNK, SUP_DIM), jnp.float32),
            pltpu.VMEM((CHUNK, SUP_DIM), jnp.float32),
            pltpu.VMEM((CHUNK, SUP_DIM), jnp.float32),
            pltpu.VMEM((BPW,), jnp.float32),
            pltpu.SemaphoreType.DMA,
            pltpu.SemaphoreType.DMA,
            pltpu.SemaphoreType.DMA,
            pltpu.SemaphoreType.DMA,
        ],
    )
    return kfn(i.reshape(NUM_WORKERS, NCHUNKS, CHUNK),
               j.reshape(NUM_WORKERS, NCHUNKS, CHUNK),
               node_emb.reshape(NUM_NODES_SUP, SUP_DIM))


NUM_NODES_SUP = 500000


def kernel(i, j, node_emb):
    return _sc_dot(i.astype(jnp.int32), j.astype(jnp.int32), node_emb)


# speedup vs baseline: 1.1447x; 1.1447x over previous
import functools

import jax
import jax.numpy as jnp
from jax import lax
from jax.experimental import pallas as pl
from jax.experimental.pallas import tpu as pltpu
from jax.experimental.pallas import tpu_sc as plsc

BATCH = 16384
EMBED_DIM = 64
SUP_DIM = 128
LANES = 16
NUM_CORES = 2
NUM_SUBCORES = 16
NUM_WORKERS = NUM_CORES * NUM_SUBCORES
BPW = BATCH // NUM_WORKERS
CHUNK = 128
NCHUNKS = BPW // CHUNK
GROUPS = CHUNK // LANES


def _dot_body(i_hbm, j_hbm, emb_hbm, out_hbm,
              idx_i, idx_j, buf_i0, buf_i1, buf_j0, buf_j1, out_v,
              si0, si1, sj0, sj1):
    c = lax.axis_index("c")
    s = lax.axis_index("s")
    wid = s * NUM_CORES + c

    pltpu.sync_copy(i_hbm.at[wid], idx_i)
    pltpu.sync_copy(j_hbm.at[wid], idx_j)

    bufs_i = (buf_i0, buf_i1)
    bufs_j = (buf_j0, buf_j1)
    sems_i = (si0, si1)
    sems_j = (sj0, sj1)

    def fire(k):
        b = k % 2
        return (pltpu.async_copy(emb_hbm.at[idx_i.at[k]], bufs_i[b], sems_i[b]),
                pltpu.async_copy(emb_hbm.at[idx_j.at[k]], bufs_j[b], sems_j[b]))

    lane = lax.broadcasted_iota(jnp.int32, (LANES,), 0)
    inflight = fire(0)

    for k in range(NCHUNKS):
        b = k % 2
        for cp in inflight:
            cp.wait()
        if k + 1 < NCHUNKS:
            inflight = fire(k + 1)
        bi, bj = bufs_i[b], bufs_j[b]
        for g in range(GROUPS):
            out_vec = jnp.zeros((LANES,), jnp.float32)
            for rl in range(LANES):
                r = g * LANES + rl
                acc = jnp.zeros((LANES,), jnp.float32)
                for d in range(EMBED_DIM // LANES):
                    vi = bi[r, pl.ds(d * LANES, LANES)]
                    vj = bj[r, pl.ds(d * LANES, LANES)]
                    acc = acc + vi * vj
                dot = jnp.sum(acc)
                out_vec = jnp.where(lane == rl, dot, out_vec)
            out_v[pl.ds(k * CHUNK + g * LANES, LANES)] = out_vec

    pltpu.sync_copy(out_v, out_hbm.at[pl.ds(wid * BPW, BPW)])


@jax.jit
def _sc_dot(i, j, node_emb):
    mesh = plsc.VectorSubcoreMesh(core_axis_name="c", subcore_axis_name="s")
    kfn = pl.kernel(
        _dot_body,
        mesh=mesh,
        compiler_params=pltpu.CompilerParams(
            needs_layout_passes=False, use_tc_tiling_on_sc=True),
        out_type=jax.ShapeDtypeStruct((BATCH,), jnp.float32),
        scratch_types=[
            pltpu.VMEM((NCHUNKS, CHUNK), jnp.int32),
            pltpu.VMEM((NCHUNKS, CHUNK), jnp.int32),
            pltpu.VMEM((CHUNK, SUP_DIM), jnp.float32),
            pltpu.VMEM((CHUNK, SUP_DIM), jnp.float32),
            pltpu.VMEM((CHUNK, SUP_DIM), jnp.float32),
            pltpu.VMEM((CHUNK, SUP_DIM), jnp.float32),
            pltpu.VMEM((BPW,), jnp.float32),
            pltpu.SemaphoreType.DMA,
            pltpu.SemaphoreType.DMA,
            pltpu.SemaphoreType.DMA,
            pltpu.SemaphoreType.DMA,
        ],
    )
    emb128 = jnp.pad(node_emb, ((0, 0), (0, SUP_DIM - EMBED_DIM)))
    return kfn(i.reshape(NUM_WORKERS, NCHUNKS, CHUNK),
               j.reshape(NUM_WORKERS, NCHUNKS, CHUNK),
               emb128)


def kernel(i, j, node_emb):
    return _sc_dot(i.astype(jnp.int32), j.astype(jnp.int32), node_emb)


# speedup vs baseline: 1.6032x; 1.4006x over previous
import functools

import jax
import jax.numpy as jnp
from jax import lax
from jax.experimental import pallas as pl
from jax.experimental.pallas import tpu as pltpu
from jax.experimental.pallas import tpu_sc as plsc

BATCH = 16384
EMBED_DIM = 64
LANES = 16
NUM_CORES = 2
NUM_SUBCORES = 16
NUM_WORKERS = NUM_CORES * NUM_SUBCORES
BPW = BATCH // NUM_WORKERS
RPB = 16
NB = BPW // RPB
SUBROWS = 8


def _fire(emb_hbm, idx_ref, blocks, sem, k):
    nvec = idx_ref[pl.ds(k * RPB, RPB)]
    base = lax.shift_right_logical(nvec, 3) * SUBROWS
    for t in range(RPB):
        pltpu.async_copy(
            emb_hbm.at[pl.ds(pl.multiple_of(base[t], SUBROWS), SUBROWS), :],
            blocks.at[t], sem)


def _drain(emb_hbm, blocks, sem):
    for t in range(RPB):
        pltpu.make_async_copy(
            emb_hbm.at[pl.ds(0, SUBROWS), :], blocks.at[t], sem).wait()


def _dot_body(i_hbm, j_hbm, emb_hbm, out_hbm,
              idx_i, idx_j, bi0, bi1, bj0, bj1, out_v,
              si0, si1, sj0, sj1):
    c = lax.axis_index("c")
    s = lax.axis_index("s")
    wid = s * NUM_CORES + c
    base_row = wid * BPW

    pltpu.sync_copy(i_hbm.at[pl.ds(base_row, BPW)], idx_i)
    pltpu.sync_copy(j_hbm.at[pl.ds(base_row, BPW)], idx_j)

    bufs_i = (bi0, bi1)
    bufs_j = (bj0, bj1)
    sems_i = (si0, si1)
    sems_j = (sj0, sj1)

    _fire(emb_hbm, idx_i, bufs_i[0], sems_i[0], 0)
    _fire(emb_hbm, idx_j, bufs_j[0], sems_j[0], 0)
    _fire(emb_hbm, idx_i, bufs_i[1], sems_i[1], 1)
    _fire(emb_hbm, idx_j, bufs_j[1], sems_j[1], 1)

    lane = lax.broadcasted_iota(jnp.int32, (LANES,), 0)

    def step(k2, _):
        for b in range(2):
            k = 2 * k2 + b
            bi, bj = bufs_i[b], bufs_j[b]
            _drain(emb_hbm, bi, sems_i[b])
            _drain(emb_hbm, bj, sems_j[b])
            nv_i = idx_i[pl.ds(k * RPB, RPB)] & 7
            nv_j = idx_j[pl.ds(k * RPB, RPB)] & 7
            out_vec = jnp.zeros((LANES,), jnp.float32)
            for t in range(RPB):
                si_t = nv_i[t]
                sj_t = nv_j[t]
                acc = jnp.zeros((LANES,), jnp.float32)
                for d in range(EMBED_DIM // LANES):
                    vi = bi[t, si_t, pl.ds(d * LANES, LANES)]
                    vj = bj[t, sj_t, pl.ds(d * LANES, LANES)]
                    acc = acc + vi * vj
                dot = jnp.sum(acc)
                out_vec = jnp.where(lane == t, dot, out_vec)
            out_v[pl.ds(k * RPB, RPB)] = out_vec

            @pl.when(k + 2 < NB)
            def _():
                _fire(emb_hbm, idx_i, bi, sems_i[b], k + 2)
                _fire(emb_hbm, idx_j, bj, sems_j[b], k + 2)
        return 0

    lax.fori_loop(0, NB // 2, step, 0)

    pltpu.sync_copy(out_v, out_hbm.at[pl.ds(base_row, BPW)])


@jax.jit
def _sc_dot(i, j, node_emb):
    mesh = plsc.VectorSubcoreMesh(core_axis_name="c", subcore_axis_name="s")
    kfn = pl.kernel(
        _dot_body,
        mesh=mesh,
        compiler_params=pltpu.CompilerParams(
            needs_layout_passes=False, use_tc_tiling_on_sc=True),
        out_type=jax.ShapeDtypeStruct((BATCH,), jnp.float32),
        scratch_types=[
            pltpu.VMEM((BPW,), jnp.int32),
            pltpu.VMEM((BPW,), jnp.int32),
            pltpu.VMEM((RPB, SUBROWS, EMBED_DIM), jnp.float32),
            pltpu.VMEM((RPB, SUBROWS, EMBED_DIM), jnp.float32),
            pltpu.VMEM((RPB, SUBROWS, EMBED_DIM), jnp.float32),
            pltpu.VMEM((RPB, SUBROWS, EMBED_DIM), jnp.float32),
            pltpu.VMEM((BPW,), jnp.float32),
            pltpu.SemaphoreType.DMA,
            pltpu.SemaphoreType.DMA,
            pltpu.SemaphoreType.DMA,
            pltpu.SemaphoreType.DMA,
        ],
    )
    return kfn(i, j, node_emb)


def kernel(i, j, node_emb):
    return _sc_dot(i.astype(jnp.int32), j.astype(jnp.int32), node_emb)
